# baseline (speedup 1.0000x reference)
import functools

import jax
import jax.numpy as jnp
from jax import lax
from jax.experimental import pallas as pl
from jax.experimental.pallas import tpu as pltpu


def _edge_const_body(ea_ref, wee_ref, bee_ref, w1c_ref, bm1_ref, out_ref):
    e = jnp.dot(ea_ref[...], wee_ref[...], preferred_element_type=jnp.float32)
    e = e + bee_ref[...]
    c = jnp.dot(e, w1c_ref[...], preferred_element_type=jnp.float32)
    out_ref[...] = c + bm1_ref[...]


def _node_pre_body(h_ref, w1a_ref, w1b_ref, a_ref, b_ref):
    h = h_ref[...]
    a_ref[...] = jnp.dot(h, w1a_ref[...], preferred_element_type=jnp.float32)
    b_ref[...] = jnp.dot(h, w1b_ref[...], preferred_element_type=jnp.float32)


def _edge_mlp_body(ad_ref, bs_ref, ce_ref, wm2_ref, bm2_ref, wm3_ref, bm3_ref,
                   out_ref):
    z1 = jnp.maximum(ad_ref[...] + bs_ref[...] + ce_ref[...], 0.0)
    z2 = jnp.dot(z1, wm2_ref[...], preferred_element_type=jnp.float32)
    z2 = jnp.maximum(z2 + bm2_ref[...], 0.0)
    z3 = jnp.dot(z2, wm3_ref[...], preferred_element_type=jnp.float32)
    out_ref[...] = jnp.maximum(z3 + bm3_ref[...], 0.0)


def _update_body(h_ref, agg_ref, wu1a_ref, wu1b_ref, bu1_ref, wu2_ref, bu2_ref,
                 wu3_ref, bu3_ref, out_ref):
    h = h_ref[...]
    u = jnp.dot(h, wu1a_ref[...], preferred_element_type=jnp.float32)
    u = u + jnp.dot(agg_ref[...], wu1b_ref[...], preferred_element_type=jnp.float32)
    u = jnp.maximum(u + bu1_ref[...], 0.0)
    u = jnp.maximum(jnp.dot(u, wu2_ref[...], preferred_element_type=jnp.float32)
                    + bu2_ref[...], 0.0)
    u = jnp.maximum(jnp.dot(u, wu3_ref[...], preferred_element_type=jnp.float32)
                    + bu3_ref[...], 0.0)
    out_ref[...] = h + u


def _vertex_enc_body(x_ref, w_ref, b_ref, out_ref):
    out_ref[...] = (jnp.dot(x_ref[...], w_ref[...],
                            preferred_element_type=jnp.float32) + b_ref[...])


def _full(shape=None):
    return pl.BlockSpec(shape, lambda i: tuple(0 for _ in shape))


def _rows(bs, d):
    return pl.BlockSpec((bs, d), lambda i: (i, 0))


def kernel(x, edge_index, edge_attr, batch, W_ve, b_ve, W_ee, b_ee,
           Wm1, bm1, Wm2, bm2, Wm3, bm3, Wu1, bu1, Wu2, bu2, Wu3, bu3):
    N, D = x.shape
    E, ED = edge_attr.shape
    H = Wm2.shape[0]
    L = 6

    src = edge_index[0]
    dst = edge_index[1]

    W1a, W1b, W1c = Wm1[:D], Wm1[D:2 * D], Wm1[2 * D:]
    Wu1a, Wu1b = Wu1[:D], Wu1[D:]

    bn = 1000 if N % 1000 == 0 else N
    be = 2000 if E % 2000 == 0 else E

    f32 = jnp.float32
    cp = pltpu.CompilerParams(dimension_semantics=("arbitrary",))

    h = pl.pallas_call(
        _vertex_enc_body,
        grid=(N // bn,),
        in_specs=[_rows(bn, D), _full((D, D)), _full((1, D))],
        out_specs=_rows(bn, D),
        out_shape=jax.ShapeDtypeStruct((N, D), f32),
        compiler_params=cp,
    )(x, W_ve, b_ve.reshape(1, D))

    c_e = pl.pallas_call(
        _edge_const_body,
        grid=(E // be,),
        in_specs=[_rows(be, ED), _full((ED, D)), _full((1, D)),
                  _full((D, H)), _full((1, H))],
        out_specs=_rows(be, H),
        out_shape=jax.ShapeDtypeStruct((E, H), f32),
        compiler_params=cp,
    )(edge_attr, W_ee, b_ee.reshape(1, D), W1c, bm1.reshape(1, H))

    node_pre = pl.pallas_call(
        _node_pre_body,
        grid=(N // bn,),
        in_specs=[_rows(bn, D), _full((D, H)), _full((D, H))],
        out_specs=[_rows(bn, H), _rows(bn, H)],
        out_shape=[jax.ShapeDtypeStruct((N, H), f32),
                   jax.ShapeDtypeStruct((N, H), f32)],
        compiler_params=cp,
    )

    edge_mlp = pl.pallas_call(
        _edge_mlp_body,
        grid=(E // be,),
        in_specs=[_rows(be, H), _rows(be, H), _rows(be, H),
                  _full((H, H)), _full((1, H)), _full((H, D)), _full((1, D))],
        out_specs=_rows(be, D),
        out_shape=jax.ShapeDtypeStruct((E, D), f32),
        compiler_params=cp,
    )

    update = pl.pallas_call(
        _update_body,
        grid=(N // bn,),
        in_specs=[_rows(bn, D), _rows(bn, D),
                  _full((D, H)), _full((D, H)), _full((1, H)),
                  _full((H, H)), _full((1, H)), _full((H, D)), _full((1, D))],
        out_specs=_rows(bn, D),
        out_shape=jax.ShapeDtypeStruct((N, D), f32),
        compiler_params=cp,
    )

    bm2r = bm2.reshape(1, H)
    bm3r = bm3.reshape(1, D)
    bu1r = bu1.reshape(1, H)
    bu2r = bu2.reshape(1, H)
    bu3r = bu3.reshape(1, D)

    for _ in range(L):
        a, b = node_pre(h, W1a, W1b)
        a_d = jnp.take(a, dst, axis=0)
        b_s = jnp.take(b, src, axis=0)
        msg = edge_mlp(a_d, b_s, c_e, Wm2, bm2r, Wm3, bm3r)
        agg = jax.ops.segment_sum(msg, dst, num_segments=N)
        h = update(h, agg, Wu1a, Wu1b, bu1r, Wu2, bu2r, Wu3, bu3r)
    return h
